# initial kernel scaffold (unmeasured)
import jax
import jax.numpy as jnp
from jax import lax
from jax.experimental import pallas as pl
from jax.experimental.pallas import tpu as pltpu

B, H, D, BS = 8, 8, 128, 16
NB = 512
NPAGES_LOCAL = 512
CP = 64
NCHUNK = NPAGES_LOCAL // CP
KEYS = CP * BS
NEG = -1e30
SCALE = D ** -0.5


def kernel(Q, K, V, bt, lens):
    lens2 = lens.reshape(B, 1)

    def body(q_ref, k_ref, v_ref, bt_ref, lens_ref, out_ref,
             m_ref, l_ref, acc_ref,
             sacc_ref, sml_ref, racc_ref, rml_ref,
             send_sems, recv_sems):
        c = pl.program_id(0)
        my_x = lax.axis_index("x")
        my_y = lax.axis_index("y")

        @pl.when(c == 0)
        def _init():
            m_ref[...] = jnp.full((B, H), NEG, jnp.float32)
            l_ref[...] = jnp.zeros((B, H), jnp.float32)
            acc_ref[...] = jnp.zeros((B, H, D), jnp.float32)

        base = my_x * NPAGES_LOCAL + c * CP
        ids = base + lax.broadcasted_iota(jnp.int32, (1, 1, CP), 2)
        j_idx = lax.broadcasted_iota(jnp.int32, (B, NB), 1)
        valid = j_idx < lens_ref[...]
        hit = (bt_ref[...][:, :, None] == ids) & valid[:, :, None]
        counts = jnp.sum(hit.astype(jnp.float32), axis=1)
        countk = jnp.broadcast_to(
            counts[:, :, None], (B, CP, BS)
        ).reshape(B, KEYS)

        s_list = []
        for h in range(H):
            q_h = q_ref[:, 0, h, :].astype(jnp.bfloat16)
            k_h = k_ref[:, :, h, :].reshape(KEYS, D).astype(jnp.bfloat16)
            s_h = lax.dot_general(
                q_h, k_h, (((1,), (1,)), ((), ())),
                preferred_element_type=jnp.float32,
            ) * SCALE
            s_list.append(s_h)
        s = jnp.stack(s_list, axis=1)
        s = jnp.where(countk[:, None, :] > 0, s, NEG)

        m_old = m_ref[...]
        m_new = jnp.maximum(m_old, jnp.max(s, axis=-1))
        p = countk[:, None, :] * jnp.exp(s - m_new[:, :, None])
        alpha = jnp.exp(m_old - m_new)
        l_ref[...] = l_ref[...] * alpha + jnp.sum(p, axis=-1)

        pv_list = []
        for h in range(H):
            v_h = v_ref[:, :, h, :].reshape(KEYS, D).astype(jnp.bfloat16)
            pv_h = lax.dot_general(
                p[:, h, :].astype(jnp.bfloat16), v_h,
                (((1,), (0,)), ((), ())),
                preferred_element_type=jnp.float32,
            )
            pv_list.append(pv_h)
        pv = jnp.stack(pv_list, axis=1)
        acc_ref[...] = acc_ref[...] * alpha[:, :, None] + pv
        m_ref[...] = m_new

        @pl.when(c == NCHUNK - 1)
        def _finish():
            sacc_ref[...] = acc_ref[...]
            sml_ref[0] = m_ref[...]
            sml_ref[1] = l_ref[...]

            partner = (1 - my_x, my_y)
            barrier_sem = pltpu.get_barrier_semaphore()
            pl.semaphore_signal(
                barrier_sem, inc=1,
                device_id=partner, device_id_type=pl.DeviceIdType.MESH,
            )
            pl.semaphore_wait(barrier_sem, 1)

            rdma_acc = pltpu.make_async_remote_copy(
                src_ref=sacc_ref, dst_ref=racc_ref,
                send_sem=send_sems.at[0], recv_sem=recv_sems.at[0],
                device_id=partner, device_id_type=pl.DeviceIdType.MESH,
            )
            rdma_ml = pltpu.make_async_remote_copy(
                src_ref=sml_ref, dst_ref=rml_ref,
                send_sem=send_sems.at[1], recv_sem=recv_sems.at[1],
                device_id=partner, device_id_type=pl.DeviceIdType.MESH,
            )
            rdma_acc.start()
            rdma_ml.start()
            rdma_acc.wait()
            rdma_ml.wait()

            m1, l1, acc1 = m_ref[...], l_ref[...], acc_ref[...]
            m2, l2, acc2 = rml_ref[0], rml_ref[1], racc_ref[...]
            m_tot = jnp.maximum(m1, m2)
            a1 = jnp.exp(m1 - m_tot)
            a2 = jnp.exp(m2 - m_tot)
            l_tot = l1 * a1 + l2 * a2
            out = (acc1 * a1[:, :, None] + acc2 * a2[:, :, None]) / l_tot[:, :, None]
            out_ref[...] = out.reshape(B, 1, H, D).astype(jnp.float32)

    return pl.pallas_call(
        body,
        grid=(NCHUNK,),
        out_shape=jax.ShapeDtypeStruct((B, 1, H, D), jnp.float32),
        in_specs=[
            pl.BlockSpec((B, 1, H, D), lambda c: (0, 0, 0, 0)),
            pl.BlockSpec((CP, BS, H, D), lambda c: (c, 0, 0, 0)),
            pl.BlockSpec((CP, BS, H, D), lambda c: (c, 0, 0, 0)),
            pl.BlockSpec((B, NB), lambda c: (0, 0)),
            pl.BlockSpec((B, 1), lambda c: (0, 0)),
        ],
        out_specs=pl.BlockSpec((B, 1, H, D), lambda c: (0, 0, 0, 0)),
        scratch_shapes=[
            pltpu.VMEM((B, H), jnp.float32),
            pltpu.VMEM((B, H), jnp.float32),
            pltpu.VMEM((B, H, D), jnp.float32),
            pltpu.VMEM((B, H, D), jnp.float32),
            pltpu.VMEM((2, B, H), jnp.float32),
            pltpu.VMEM((B, H, D), jnp.float32),
            pltpu.VMEM((2, B, H), jnp.float32),
            pltpu.SemaphoreType.DMA((2,)),
            pltpu.SemaphoreType.DMA((2,)),
        ],
        compiler_params=pltpu.CompilerParams(
            collective_id=0,
            dimension_semantics=("arbitrary",),
        ),
    )(Q, K, V, bt, lens2)


# baseline (device time: 81024 ns/iter reference)
import jax
import jax.numpy as jnp
from jax import lax
from jax.experimental import pallas as pl
from jax.experimental.pallas import tpu as pltpu

B, H, D, BS = 8, 8, 128, 16
HB = H * B
NB = 512
NPAGES_LOCAL = 512
CP = 64
NCHUNK = NPAGES_LOCAL // CP
KEYS = CP * BS
NEG = -1e30
SCALE = D ** -0.5


def kernel(Q, K, V, bt, lens):
    lens2 = lens.reshape(B, 1)

    def body(q_ref, k_ref, v_ref, bt_ref, lens_ref, out_ref,
             m_ref, l_ref, acc_ref,
             sacc_ref, sml_ref, racc_ref, rml_ref,
             send_sems, recv_sems):
        c = pl.program_id(0)
        my_x = lax.axis_index("x")
        my_y = lax.axis_index("y")

        @pl.when(c == 0)
        def _init():
            m_ref[...] = jnp.full((HB, 1), NEG, jnp.float32)
            l_ref[...] = jnp.zeros((HB, 1), jnp.float32)
            acc_ref[...] = jnp.zeros((HB, D), jnp.float32)

        j_idx = lax.broadcasted_iota(jnp.int32, (B, NB), 1)
        btm = jnp.where(j_idx < lens_ref[...], bt_ref[...], -1)
        base = my_x * NPAGES_LOCAL + c * CP
        ids = base + lax.broadcasted_iota(jnp.int32, (B, CP, NB), 1)
        hit = (btm[:, None, :] == ids).astype(jnp.float32)
        counts = jnp.sum(hit, axis=2)

        pg_of_key = lax.broadcasted_iota(jnp.int32, (CP, KEYS), 1) // BS
        pg_row = lax.broadcasted_iota(jnp.int32, (CP, KEYS), 0)
        expand = (pg_of_key == pg_row).astype(jnp.float32)
        countk = lax.dot_general(
            counts, expand, (((1,), (0,)), ((), ())),
            preferred_element_type=jnp.float32,
        )
        countk_hb = jnp.broadcast_to(countk[None], (H, B, KEYS)).reshape(HB, KEYS)

        s_list = []
        for h in range(H):
            q_h = q_ref[:, 0, h, :].astype(jnp.bfloat16)
            k_h = k_ref[:, :, h, :].reshape(KEYS, D).astype(jnp.bfloat16)
            s_h = lax.dot_general(
                q_h, k_h, (((1,), (1,)), ((), ())),
                preferred_element_type=jnp.float32,
            )
            s_list.append(s_h)
        s = jnp.stack(s_list, axis=0).reshape(HB, KEYS) * SCALE
        s = jnp.where(countk_hb > 0, s, NEG)

        m_old = m_ref[...]
        m_new = jnp.maximum(m_old, jnp.max(s, axis=1, keepdims=True))
        p = countk_hb * jnp.exp(s - m_new)
        alpha = jnp.exp(m_old - m_new)
        l_ref[...] = l_ref[...] * alpha + jnp.sum(p, axis=1, keepdims=True)

        p16 = p.astype(jnp.bfloat16)
        pv_list = []
        for h in range(H):
            v_h = v_ref[:, :, h, :].reshape(KEYS, D).astype(jnp.bfloat16)
            pv_h = lax.dot_general(
                p16[h * B:(h + 1) * B, :], v_h,
                (((1,), (0,)), ((), ())),
                preferred_element_type=jnp.float32,
            )
            pv_list.append(pv_h)
        pv = jnp.stack(pv_list, axis=0).reshape(HB, D)
        acc_ref[...] = acc_ref[...] * alpha + pv
        m_ref[...] = m_new

        @pl.when(c == NCHUNK - 1)
        def _finish():
            sacc_ref[...] = acc_ref[...]
            sml_ref[0] = m_ref[...]
            sml_ref[1] = l_ref[...]

            partner = (1 - my_x, my_y)
            barrier_sem = pltpu.get_barrier_semaphore()
            pl.semaphore_signal(
                barrier_sem, inc=1,
                device_id=partner, device_id_type=pl.DeviceIdType.MESH,
            )
            pl.semaphore_wait(barrier_sem, 1)

            rdma_acc = pltpu.make_async_remote_copy(
                src_ref=sacc_ref, dst_ref=racc_ref,
                send_sem=send_sems.at[0], recv_sem=recv_sems.at[0],
                device_id=partner, device_id_type=pl.DeviceIdType.MESH,
            )
            rdma_ml = pltpu.make_async_remote_copy(
                src_ref=sml_ref, dst_ref=rml_ref,
                send_sem=send_sems.at[1], recv_sem=recv_sems.at[1],
                device_id=partner, device_id_type=pl.DeviceIdType.MESH,
            )
            rdma_acc.start()
            rdma_ml.start()
            rdma_acc.wait()
            rdma_ml.wait()

            m1, l1, acc1 = m_ref[...], l_ref[...], acc_ref[...]
            m2, l2, acc2 = rml_ref[0], rml_ref[1], racc_ref[...]
            m_tot = jnp.maximum(m1, m2)
            a1 = jnp.exp(m1 - m_tot)
            a2 = jnp.exp(m2 - m_tot)
            l_tot = l1 * a1 + l2 * a2
            final = (acc1 * a1 + acc2 * a2) / l_tot
            for h in range(H):
                out_ref[:, 0, h, :] = final[h * B:(h + 1) * B, :]

    return pl.pallas_call(
        body,
        grid=(NCHUNK,),
        out_shape=jax.ShapeDtypeStruct((B, 1, H, D), jnp.float32),
        in_specs=[
            pl.BlockSpec((B, 1, H, D), lambda c: (0, 0, 0, 0)),
            pl.BlockSpec((CP, BS, H, D), lambda c: (c, 0, 0, 0)),
            pl.BlockSpec((CP, BS, H, D), lambda c: (c, 0, 0, 0)),
            pl.BlockSpec((B, NB), lambda c: (0, 0)),
            pl.BlockSpec((B, 1), lambda c: (0, 0)),
        ],
        out_specs=pl.BlockSpec((B, 1, H, D), lambda c: (0, 0, 0, 0)),
        scratch_shapes=[
            pltpu.VMEM((HB, 1), jnp.float32),
            pltpu.VMEM((HB, 1), jnp.float32),
            pltpu.VMEM((HB, D), jnp.float32),
            pltpu.VMEM((HB, D), jnp.float32),
            pltpu.VMEM((2, HB, 1), jnp.float32),
            pltpu.VMEM((HB, D), jnp.float32),
            pltpu.VMEM((2, HB, 1), jnp.float32),
            pltpu.SemaphoreType.DMA((2,)),
            pltpu.SemaphoreType.DMA((2,)),
        ],
        compiler_params=pltpu.CompilerParams(
            collective_id=0,
            dimension_semantics=("arbitrary",),
        ),
    )(Q, K, V, bt, lens2)


# device time: 75931 ns/iter; 1.0671x vs baseline; 1.0671x over previous
import jax
import jax.numpy as jnp
from jax import lax
from jax.experimental import pallas as pl
from jax.experimental.pallas import tpu as pltpu

B, H, D, BS = 8, 8, 128, 16
HB = H * B
HD = H * D
NB = 512
NPAGES_LOCAL = 512
CP = 64
NCHUNK = NPAGES_LOCAL // CP
KEYS = CP * BS
NEG = -1e30
SCALE = D ** -0.5


def kernel(Q, K, V, bt, lens):
    lens2 = lens.reshape(B, 1)
    K2 = K.reshape(NPAGES_LOCAL * BS, HD)
    V2 = V.reshape(NPAGES_LOCAL * BS, HD)

    def body(q_ref, k_ref, v_ref, bt_ref, lens_ref, out_ref,
             qm_ref, m_ref, l_ref, acc_ref,
             sacc_ref, sml_ref, racc_ref, rml_ref,
             send_sems, recv_sems):
        c = pl.program_id(0)
        my_x = lax.axis_index("x")
        my_y = lax.axis_index("y")

        @pl.when(c == 0)
        def _init():
            m_ref[...] = jnp.full((HB, 1), NEG, jnp.float32)
            l_ref[...] = jnp.zeros((HB, 1), jnp.float32)
            acc_ref[...] = jnp.zeros((HB, D), jnp.float32)
            qm_ref[...] = jnp.zeros((HB, HD), jnp.bfloat16)
            for h in range(H):
                qm_ref[h * B:(h + 1) * B, h * D:(h + 1) * D] = (
                    q_ref[:, 0, h, :].astype(jnp.bfloat16))

        j_idx = lax.broadcasted_iota(jnp.int32, (B, NB), 1)
        btm = jnp.where(j_idx < lens_ref[...], bt_ref[...], -1)
        base = my_x * NPAGES_LOCAL + c * CP
        ids = base + lax.broadcasted_iota(jnp.int32, (B, CP, NB), 1)
        hit = (btm[:, None, :] == ids).astype(jnp.float32)
        counts = jnp.sum(hit, axis=2)

        pg_of_key = lax.broadcasted_iota(jnp.int32, (CP, KEYS), 1) // BS
        pg_row = lax.broadcasted_iota(jnp.int32, (CP, KEYS), 0)
        expand = (pg_of_key == pg_row).astype(jnp.float32)
        countk = lax.dot_general(
            counts, expand, (((1,), (0,)), ((), ())),
            preferred_element_type=jnp.float32,
        )
        countk_hb = jnp.broadcast_to(countk[None], (H, B, KEYS)).reshape(HB, KEYS)

        k16 = k_ref[...].astype(jnp.bfloat16)
        s = lax.dot_general(
            qm_ref[...], k16, (((1,), (1,)), ((), ())),
            preferred_element_type=jnp.float32,
        ) * SCALE
        s = jnp.where(countk_hb > 0, s, NEG)

        m_old = m_ref[...]
        m_new = jnp.maximum(m_old, jnp.max(s, axis=1, keepdims=True))
        p = countk_hb * jnp.exp(s - m_new)
        alpha = jnp.exp(m_old - m_new)
        l_ref[...] = l_ref[...] * alpha + jnp.sum(p, axis=1, keepdims=True)

        v16 = v_ref[...].astype(jnp.bfloat16)
        pv_big = lax.dot_general(
            p.astype(jnp.bfloat16), v16, (((1,), (0,)), ((), ())),
            preferred_element_type=jnp.float32,
        )
        pv = jnp.concatenate(
            [pv_big[h * B:(h + 1) * B, h * D:(h + 1) * D] for h in range(H)],
            axis=0,
        )
        acc_ref[...] = acc_ref[...] * alpha + pv
        m_ref[...] = m_new

        @pl.when(c == NCHUNK - 1)
        def _finish():
            sacc_ref[...] = acc_ref[...]
            sml_ref[0] = m_ref[...]
            sml_ref[1] = l_ref[...]

            partner = (1 - my_x, my_y)
            barrier_sem = pltpu.get_barrier_semaphore()
            pl.semaphore_signal(
                barrier_sem, inc=1,
                device_id=partner, device_id_type=pl.DeviceIdType.MESH,
            )
            pl.semaphore_wait(barrier_sem, 1)

            rdma_acc = pltpu.make_async_remote_copy(
                src_ref=sacc_ref, dst_ref=racc_ref,
                send_sem=send_sems.at[0], recv_sem=recv_sems.at[0],
                device_id=partner, device_id_type=pl.DeviceIdType.MESH,
            )
            rdma_ml = pltpu.make_async_remote_copy(
                src_ref=sml_ref, dst_ref=rml_ref,
                send_sem=send_sems.at[1], recv_sem=recv_sems.at[1],
                device_id=partner, device_id_type=pl.DeviceIdType.MESH,
            )
            rdma_acc.start()
            rdma_ml.start()
            rdma_acc.wait()
            rdma_ml.wait()

            m1, l1, acc1 = m_ref[...], l_ref[...], acc_ref[...]
            m2, l2, acc2 = rml_ref[0], rml_ref[1], racc_ref[...]
            m_tot = jnp.maximum(m1, m2)
            a1 = jnp.exp(m1 - m_tot)
            a2 = jnp.exp(m2 - m_tot)
            l_tot = l1 * a1 + l2 * a2
            final = (acc1 * a1 + acc2 * a2) / l_tot
            for h in range(H):
                out_ref[:, 0, h, :] = final[h * B:(h + 1) * B, :]

    return pl.pallas_call(
        body,
        grid=(NCHUNK,),
        out_shape=jax.ShapeDtypeStruct((B, 1, H, D), jnp.float32),
        in_specs=[
            pl.BlockSpec((B, 1, H, D), lambda c: (0, 0, 0, 0)),
            pl.BlockSpec((KEYS, HD), lambda c: (c, 0)),
            pl.BlockSpec((KEYS, HD), lambda c: (c, 0)),
            pl.BlockSpec((B, NB), lambda c: (0, 0)),
            pl.BlockSpec((B, 1), lambda c: (0, 0)),
        ],
        out_specs=pl.BlockSpec((B, 1, H, D), lambda c: (0, 0, 0, 0)),
        scratch_shapes=[
            pltpu.VMEM((HB, HD), jnp.bfloat16),
            pltpu.VMEM((HB, 1), jnp.float32),
            pltpu.VMEM((HB, 1), jnp.float32),
            pltpu.VMEM((HB, D), jnp.float32),
            pltpu.VMEM((HB, D), jnp.float32),
            pltpu.VMEM((2, HB, 1), jnp.float32),
            pltpu.VMEM((HB, D), jnp.float32),
            pltpu.VMEM((2, HB, 1), jnp.float32),
            pltpu.SemaphoreType.DMA((2,)),
            pltpu.SemaphoreType.DMA((2,)),
        ],
        compiler_params=pltpu.CompilerParams(
            collective_id=0,
            dimension_semantics=("arbitrary",),
        ),
    )(Q, K2, V2, bt, lens2)


# device time: 32067 ns/iter; 2.5267x vs baseline; 2.3679x over previous
import jax
import jax.numpy as jnp
from jax import lax
from jax.experimental import pallas as pl
from jax.experimental.pallas import tpu as pltpu

B, H, D, BS = 8, 8, 128, 16
HB = H * B
HD = H * D
NB = 512
NPAGES_LOCAL = 512
CP = 64
NCHUNK = NPAGES_LOCAL // CP
KEYS = CP * BS
NEG = -1e30
SCALE = D ** -0.5


def kernel(Q, K, V, bt, lens):
    lens2 = lens.reshape(B, 1)

    def body(q_ref, k_ref, v_ref, bt_ref, lens_ref, out_ref,
             qm_ref, m_ref, l_ref, acc_ref,
             sacc_ref, sml_ref, racc_ref, rml_ref,
             send_sems, recv_sems):
        c = pl.program_id(0)
        my_x = lax.axis_index("x")
        my_y = lax.axis_index("y")

        @pl.when(c == 0)
        def _init():
            m_ref[...] = jnp.full((HB, 1), NEG, jnp.float32)
            l_ref[...] = jnp.zeros((HB, 1), jnp.float32)
            acc_ref[...] = jnp.zeros((HB, D), jnp.float32)
            qm_ref[...] = jnp.zeros((HB, HD), jnp.bfloat16)
            for h in range(H):
                qm_ref[h * B:(h + 1) * B, h * D:(h + 1) * D] = (
                    q_ref[:, 0, h, :].astype(jnp.bfloat16))

        j_idx = lax.broadcasted_iota(jnp.int32, (B, NB), 1)
        btm = jnp.where(j_idx < lens_ref[...], bt_ref[...], -1)
        base = my_x * NPAGES_LOCAL + c * CP
        ids = base + lax.broadcasted_iota(jnp.int32, (B, CP, NB), 1)
        hit = (btm[:, None, :] == ids).astype(jnp.float32)
        counts = jnp.sum(hit, axis=2)

        pg_of_key = lax.broadcasted_iota(jnp.int32, (CP, KEYS), 1) // BS
        pg_row = lax.broadcasted_iota(jnp.int32, (CP, KEYS), 0)
        expand = (pg_of_key == pg_row).astype(jnp.float32)
        countk = lax.dot_general(
            counts, expand, (((1,), (0,)), ((), ())),
            preferred_element_type=jnp.float32,
        )
        countk_hb = jnp.broadcast_to(countk[None], (H, B, KEYS)).reshape(HB, KEYS)

        k16 = k_ref[...].reshape(KEYS, HD).astype(jnp.bfloat16)
        s = lax.dot_general(
            qm_ref[...], k16, (((1,), (1,)), ((), ())),
            preferred_element_type=jnp.float32,
        ) * SCALE
        s = jnp.where(countk_hb > 0, s, NEG)

        m_old = m_ref[...]
        m_new = jnp.maximum(m_old, jnp.max(s, axis=1, keepdims=True))
        p = countk_hb * jnp.exp(s - m_new)
        alpha = jnp.exp(m_old - m_new)
        l_ref[...] = l_ref[...] * alpha + jnp.sum(p, axis=1, keepdims=True)

        v16 = v_ref[...].reshape(KEYS, HD).astype(jnp.bfloat16)
        pv_big = lax.dot_general(
            p.astype(jnp.bfloat16), v16, (((1,), (0,)), ((), ())),
            preferred_element_type=jnp.float32,
        )
        pv = jnp.concatenate(
            [pv_big[h * B:(h + 1) * B, h * D:(h + 1) * D] for h in range(H)],
            axis=0,
        )
        acc_ref[...] = acc_ref[...] * alpha + pv
        m_ref[...] = m_new

        @pl.when(c == NCHUNK - 1)
        def _finish():
            sacc_ref[...] = acc_ref[...]
            sml_ref[0] = m_ref[...]
            sml_ref[1] = l_ref[...]

            partner = (1 - my_x, my_y)
            barrier_sem = pltpu.get_barrier_semaphore()
            pl.semaphore_signal(
                barrier_sem, inc=1,
                device_id=partner, device_id_type=pl.DeviceIdType.MESH,
            )
            pl.semaphore_wait(barrier_sem, 1)

            rdma_acc = pltpu.make_async_remote_copy(
                src_ref=sacc_ref, dst_ref=racc_ref,
                send_sem=send_sems.at[0], recv_sem=recv_sems.at[0],
                device_id=partner, device_id_type=pl.DeviceIdType.MESH,
            )
            rdma_ml = pltpu.make_async_remote_copy(
                src_ref=sml_ref, dst_ref=rml_ref,
                send_sem=send_sems.at[1], recv_sem=recv_sems.at[1],
                device_id=partner, device_id_type=pl.DeviceIdType.MESH,
            )
            rdma_acc.start()
            rdma_ml.start()
            rdma_acc.wait()
            rdma_ml.wait()

            m1, l1, acc1 = m_ref[...], l_ref[...], acc_ref[...]
            m2, l2, acc2 = rml_ref[0], rml_ref[1], racc_ref[...]
            m_tot = jnp.maximum(m1, m2)
            a1 = jnp.exp(m1 - m_tot)
            a2 = jnp.exp(m2 - m_tot)
            l_tot = l1 * a1 + l2 * a2
            final = (acc1 * a1 + acc2 * a2) / l_tot
            for h in range(H):
                out_ref[:, 0, h, :] = final[h * B:(h + 1) * B, :]

    return pl.pallas_call(
        body,
        grid=(NCHUNK,),
        out_shape=jax.ShapeDtypeStruct((B, 1, H, D), jnp.float32),
        in_specs=[
            pl.BlockSpec((B, 1, H, D), lambda c: (0, 0, 0, 0)),
            pl.BlockSpec((CP, BS, H, D), lambda c: (c, 0, 0, 0)),
            pl.BlockSpec((CP, BS, H, D), lambda c: (c, 0, 0, 0)),
            pl.BlockSpec((B, NB), lambda c: (0, 0)),
            pl.BlockSpec((B, 1), lambda c: (0, 0)),
        ],
        out_specs=pl.BlockSpec((B, 1, H, D), lambda c: (0, 0, 0, 0)),
        scratch_shapes=[
            pltpu.VMEM((HB, HD), jnp.bfloat16),
            pltpu.VMEM((HB, 1), jnp.float32),
            pltpu.VMEM((HB, 1), jnp.float32),
            pltpu.VMEM((HB, D), jnp.float32),
            pltpu.VMEM((HB, D), jnp.float32),
            pltpu.VMEM((2, HB, 1), jnp.float32),
            pltpu.VMEM((HB, D), jnp.float32),
            pltpu.VMEM((2, HB, 1), jnp.float32),
            pltpu.SemaphoreType.DMA((2,)),
            pltpu.SemaphoreType.DMA((2,)),
        ],
        compiler_params=pltpu.CompilerParams(
            collective_id=0,
            dimension_semantics=("arbitrary",),
        ),
    )(Q, K, V, bt, lens2)


# device time: 24877 ns/iter; 3.2570x vs baseline; 1.2890x over previous
import jax
import jax.numpy as jnp
from jax import lax
from jax.experimental import pallas as pl
from jax.experimental.pallas import tpu as pltpu

B, H, D, BS = 8, 8, 128, 16
HB = H * B
HD = H * D
NB = 512
NPAGES_LOCAL = 512
PAGES_Y = NPAGES_LOCAL // 2
CP = 64
NCHUNK = PAGES_Y // CP
KEYS = CP * BS
NEG = -1e30
SCALE = D ** -0.5


def kernel(Q, K, V, bt, lens):
    lens2 = lens.reshape(B, 1)

    def body(q_ref, k_ref, v_ref, bt_ref, lens_ref, out_ref,
             qm_ref, m_ref, l_ref, acc_ref,
             sacc_ref, sml_ref, racc_ref, rml_ref,
             sacc2_ref, sml2_ref, racc2_ref, rml2_ref,
             send_sems, recv_sems):
        c = pl.program_id(0)
        my_x = lax.axis_index("x")
        my_y = lax.axis_index("y")

        @pl.when(c == 0)
        def _init():
            m_ref[...] = jnp.full((HB, 1), NEG, jnp.float32)
            l_ref[...] = jnp.zeros((HB, 1), jnp.float32)
            acc_ref[...] = jnp.zeros((HB, D), jnp.float32)
            qm_ref[...] = jnp.zeros((HB, HD), jnp.bfloat16)
            for h in range(H):
                qm_ref[h * B:(h + 1) * B, h * D:(h + 1) * D] = (
                    q_ref[:, 0, h, :].astype(jnp.bfloat16))

        j_idx = lax.broadcasted_iota(jnp.int32, (B, NB), 1)
        btm = jnp.where(j_idx < lens_ref[...], bt_ref[...], -1)
        base = my_x * NPAGES_LOCAL + my_y * PAGES_Y + c * CP
        ids = base + lax.broadcasted_iota(jnp.int32, (B, CP, NB), 1)
        hit = (btm[:, None, :] == ids).astype(jnp.float32)
        counts = jnp.sum(hit, axis=2)

        pg_of_key = lax.broadcasted_iota(jnp.int32, (CP, KEYS), 1) // BS
        pg_row = lax.broadcasted_iota(jnp.int32, (CP, KEYS), 0)
        expand = (pg_of_key == pg_row).astype(jnp.float32)
        countk = lax.dot_general(
            counts, expand, (((1,), (0,)), ((), ())),
            preferred_element_type=jnp.float32,
        )
        countk_hb = jnp.broadcast_to(countk[None], (H, B, KEYS)).reshape(HB, KEYS)

        k16 = k_ref[...].reshape(KEYS, HD).astype(jnp.bfloat16)
        s = lax.dot_general(
            qm_ref[...], k16, (((1,), (1,)), ((), ())),
            preferred_element_type=jnp.float32,
        ) * SCALE
        s = jnp.where(countk_hb > 0, s, NEG)

        m_old = m_ref[...]
        m_new = jnp.maximum(m_old, jnp.max(s, axis=1, keepdims=True))
        p = countk_hb * jnp.exp(s - m_new)
        alpha = jnp.exp(m_old - m_new)
        l_ref[...] = l_ref[...] * alpha + jnp.sum(p, axis=1, keepdims=True)

        v16 = v_ref[...].reshape(KEYS, HD).astype(jnp.bfloat16)
        pv_big = lax.dot_general(
            p.astype(jnp.bfloat16), v16, (((1,), (0,)), ((), ())),
            preferred_element_type=jnp.float32,
        )
        pv = jnp.concatenate(
            [pv_big[h * B:(h + 1) * B, h * D:(h + 1) * D] for h in range(H)],
            axis=0,
        )
        acc_ref[...] = acc_ref[...] * alpha + pv
        m_ref[...] = m_new

        @pl.when(c == NCHUNK - 1)
        def _finish():
            sacc_ref[...] = acc_ref[...]
            sml_ref[0] = m_ref[...]
            sml_ref[1] = l_ref[...]

            partner_y = (my_x, 1 - my_y)
            partner_x = (1 - my_x, my_y)
            barrier_sem = pltpu.get_barrier_semaphore()
            for partner in (partner_y, partner_x):
                pl.semaphore_signal(
                    barrier_sem, inc=1,
                    device_id=partner, device_id_type=pl.DeviceIdType.MESH,
                )
            pl.semaphore_wait(barrier_sem, 2)

            rdma_acc = pltpu.make_async_remote_copy(
                src_ref=sacc_ref, dst_ref=racc_ref,
                send_sem=send_sems.at[0], recv_sem=recv_sems.at[0],
                device_id=partner_y, device_id_type=pl.DeviceIdType.MESH,
            )
            rdma_ml = pltpu.make_async_remote_copy(
                src_ref=sml_ref, dst_ref=rml_ref,
                send_sem=send_sems.at[1], recv_sem=recv_sems.at[1],
                device_id=partner_y, device_id_type=pl.DeviceIdType.MESH,
            )
            rdma_acc.start()
            rdma_ml.start()
            rdma_acc.wait()
            rdma_ml.wait()

            m1, l1, acc1 = m_ref[...], l_ref[...], acc_ref[...]
            m2, l2, acc2 = rml_ref[0], rml_ref[1], racc_ref[...]
            m12 = jnp.maximum(m1, m2)
            a1 = jnp.exp(m1 - m12)
            a2 = jnp.exp(m2 - m12)
            l12 = l1 * a1 + l2 * a2
            acc12 = acc1 * a1 + acc2 * a2

            sacc2_ref[...] = acc12
            sml2_ref[0] = m12
            sml2_ref[1] = l12
            rdma_acc2 = pltpu.make_async_remote_copy(
                src_ref=sacc2_ref, dst_ref=racc2_ref,
                send_sem=send_sems.at[2], recv_sem=recv_sems.at[2],
                device_id=partner_x, device_id_type=pl.DeviceIdType.MESH,
            )
            rdma_ml2 = pltpu.make_async_remote_copy(
                src_ref=sml2_ref, dst_ref=rml2_ref,
                send_sem=send_sems.at[3], recv_sem=recv_sems.at[3],
                device_id=partner_x, device_id_type=pl.DeviceIdType.MESH,
            )
            rdma_acc2.start()
            rdma_ml2.start()
            rdma_acc2.wait()
            rdma_ml2.wait()

            m3, l3, acc3 = rml2_ref[0], rml2_ref[1], racc2_ref[...]
            m_tot = jnp.maximum(m12, m3)
            b1 = jnp.exp(m12 - m_tot)
            b2 = jnp.exp(m3 - m_tot)
            l_tot = l12 * b1 + l3 * b2
            final = (acc12 * b1 + acc3 * b2) / l_tot
            for h in range(H):
                out_ref[:, 0, h, :] = final[h * B:(h + 1) * B, :]

    return pl.pallas_call(
        body,
        grid=(NCHUNK,),
        out_shape=jax.ShapeDtypeStruct((B, 1, H, D), jnp.float32),
        in_specs=[
            pl.BlockSpec((B, 1, H, D), lambda c: (0, 0, 0, 0)),
            pl.BlockSpec(
                (CP, BS, H, D),
                lambda c: (lax.axis_index("y") * NCHUNK + c, 0, 0, 0),
            ),
            pl.BlockSpec(
                (CP, BS, H, D),
                lambda c: (lax.axis_index("y") * NCHUNK + c, 0, 0, 0),
            ),
            pl.BlockSpec((B, NB), lambda c: (0, 0)),
            pl.BlockSpec((B, 1), lambda c: (0, 0)),
        ],
        out_specs=pl.BlockSpec((B, 1, H, D), lambda c: (0, 0, 0, 0)),
        scratch_shapes=[
            pltpu.VMEM((HB, HD), jnp.bfloat16),
            pltpu.VMEM((HB, 1), jnp.float32),
            pltpu.VMEM((HB, 1), jnp.float32),
            pltpu.VMEM((HB, D), jnp.float32),
            pltpu.VMEM((HB, D), jnp.float32),
            pltpu.VMEM((2, HB, 1), jnp.float32),
            pltpu.VMEM((HB, D), jnp.float32),
            pltpu.VMEM((2, HB, 1), jnp.float32),
            pltpu.VMEM((HB, D), jnp.float32),
            pltpu.VMEM((2, HB, 1), jnp.float32),
            pltpu.VMEM((HB, D), jnp.float32),
            pltpu.VMEM((2, HB, 1), jnp.float32),
            pltpu.SemaphoreType.DMA((4,)),
            pltpu.SemaphoreType.DMA((4,)),
        ],
        compiler_params=pltpu.CompilerParams(
            collective_id=0,
            dimension_semantics=("arbitrary",),
        ),
    )(Q, K, V, bt, lens2)
